# baseline (speedup 1.0000x reference)
import functools

import jax
import jax.numpy as jnp
from jax import lax
from jax.experimental import pallas as pl
from jax.experimental.pallas import tpu as pltpu
from jax.experimental.pallas import tpu_sc as plsc

_LANES = 16
_NW = 32


def _vrsqrt(x):
    i = plsc.bitcast(x, jnp.int32)
    i = jnp.int32(0x5F3759DF) - (i >> 1)
    y = plsc.bitcast(i, jnp.float32)
    for _ in range(3):
        y = y * (1.5 - 0.5 * x * y * y)
    return y


@functools.lru_cache(maxsize=None)
def _build(ntok, hid, seq, ncomb, ntypes):
    ch = 2 * seq
    assert ch % _LANES == 0
    ngrp = ch // _LANES
    assert ntok % (_NW * ch) == 0
    tpw = ntok // _NW
    nch = tpw // ch
    splits = []
    off = 0
    while off < ch:
        w = min(128, ch - off)
        splits.append((off, w))
        off += w
    mesh = plsc.VectorSubcoreMesh(core_axis_name="c", subcore_axis_name="s")

    @functools.partial(
        pl.kernel,
        mesh=mesh,
        compiler_params=pltpu.CompilerParams(needs_layout_passes=False),
        out_type=jax.ShapeDtypeStruct((ntok, hid), jnp.float32),
        scratch_types=[
            pltpu.VMEM((ch,), jnp.int32),
            pltpu.VMEM((ch,), jnp.int32),
            pltpu.VMEM((ch, hid), jnp.float32),
            pltpu.VMEM((ncomb, hid), jnp.float32),
            pltpu.VMEM((2, hid), jnp.float32),
            pltpu.SemaphoreType.DMA,
        ],
    )
    def emb_kernel(ids_hbm, tt_hbm, table_hbm, comb_hbm, gb_hbm, out_hbm,
                   idx_v, tt_v, rows_v, comb_v, gb_v, sem):
        wid = lax.axis_index("s") * 2 + lax.axis_index("c")
        base = wid * tpw
        pltpu.sync_copy(comb_hbm, comb_v)
        pltpu.sync_copy(gb_hbm, gb_v)
        gvecs = [gb_v[0, pl.ds(_LANES * j, _LANES)] for j in range(hid // _LANES)]
        bvecs = [gb_v[1, pl.ds(_LANES * j, _LANES)] for j in range(hid // _LANES)]
        lane_iota = lax.iota(jnp.int32, _LANES)

        def chunk_body(c, carry):
            cb = base + c * ch
            pltpu.sync_copy(ids_hbm.at[pl.ds(cb, ch)], idx_v)
            pltpu.sync_copy(tt_hbm.at[pl.ds(cb, ch)], tt_v)
            copies = [
                pltpu.async_copy(
                    table_hbm.at[idx_v.at[pl.ds(soff, sw)]],
                    rows_v.at[pl.ds(soff, sw)],
                    sem,
                )
                for soff, sw in splits
            ]
            for cp in copies:
                cp.wait()

            def grp_body(g, gcarry):
                ridx = g * _LANES + lane_iota
                tt16 = tt_v[pl.ds(g * _LANES, _LANES)]
                ci = (ridx % seq) * ntypes + tt16
                s = jnp.zeros((_LANES,), jnp.float32)
                sq = jnp.zeros((_LANES,), jnp.float32)
                for h in range(hid):
                    col = jnp.full((_LANES,), h, jnp.int32)
                    v = (plsc.load_gather(rows_v, [ridx, col])
                         + plsc.load_gather(comb_v, [ci, col]))
                    plsc.store_scatter(rows_v, [ridx, col], v)
                    s = s + v
                    sq = sq + v * v
                mean = s * (1.0 / hid)
                r = _vrsqrt(sq * (1.0 / hid) - mean * mean + 1e-5)
                for h in range(hid):
                    col = jnp.full((_LANES,), h, jnp.int32)
                    v = plsc.load_gather(rows_v, [ridx, col])
                    gh = gvecs[h // _LANES][h % _LANES]
                    bh = bvecs[h // _LANES][h % _LANES]
                    v = (v - mean) * r * gh + bh
                    plsc.store_scatter(rows_v, [ridx, col], v)
                return gcarry

            lax.fori_loop(0, ngrp, grp_body, 0)
            pltpu.sync_copy(rows_v, out_hbm.at[pl.ds(cb, ch)])
            return carry

        lax.fori_loop(0, nch, chunk_body, 0)

    return emb_kernel


def kernel(input_ids, token_type_ids, token_table, type_table, gamma, beta, pos_enc):
    b, l = input_ids.shape
    hid = token_table.shape[1]
    ntypes = type_table.shape[0]
    ntok = b * l
    ids = input_ids.reshape(ntok)
    tt = token_type_ids.reshape(ntok)
    comb = (pos_enc[:l, None, :] + type_table[None, :, :]).reshape(l * ntypes, hid)
    gb = jnp.stack([gamma, beta])
    emb = _build(ntok, hid, l, l * ntypes, ntypes)
    out = emb(ids, tt, token_table, comb, gb)
    return out.reshape(b, l, hid)
